# Initial kernel scaffold; baseline (speedup 1.0000x reference)
import jax
import jax.numpy as jnp
from jax.experimental import pallas as pl


def kernel(X, edge_index, W_in, b_in, W_hid, b_hid, W_out, b_out):
    raise NotImplementedError("write your pallas kernel here")


# speedup vs baseline: 9.8277x; 9.8277x over previous
import functools

import jax
import jax.numpy as jnp
from jax import lax
from jax.experimental import pallas as pl
from jax.experimental.pallas import tpu as pltpu
from jax.experimental.pallas import tpu_sc as plsc

NC = 2
NS = 16
NW = NC * NS
CH = 128

_f32 = jnp.float32


def _sc_mesh():
  return plsc.VectorSubcoreMesh(core_axis_name="c", subcore_axis_name="s")


def _row_partition(N):
  rpt = ((N // NS) + 7) // 8 * 8
  last = N - rpt * (NS - 1)
  assert 0 < last <= rpt and last % 8 == 0
  return rpt, last


def _per_tile_rows(sid, rpt, last, fn):
  @pl.when(sid < NS - 1)
  def _():
    fn(pl.multiple_of(sid * rpt, 8), rpt)

  @pl.when(sid == NS - 1)
  def _():
    fn((NS - 1) * rpt, last)


def _make_deg_kernel(N, E, D):
  n_chunks = E // CH
  base = n_chunks // NW
  extra = n_chunks % NW
  rpt, last = _row_partition(N)

  @functools.partial(
      pl.kernel,
      out_type=jax.ShapeDtypeStruct((NC, N, D), _f32),
      mesh=_sc_mesh(),
      scratch_types=[
          pltpu.VMEM((CH,), jnp.int32),
          pltpu.VMEM((CH,), jnp.int32),
          pltpu.VMEM((CH, D), _f32),
          pltpu.VMEM((CH, D), _f32),
          pltpu.VMEM_SHARED((N, D), _f32),
      ],
  )
  def deg_kernel(src_hbm, dst_hbm, bsrc_hbm, bdst_hbm, zeros_hbm, out_hbm,
                 isrc_v, idst_v, bsrc_v, bdst_v, acc_sh):
    cid = lax.axis_index("c")
    sid = lax.axis_index("s")
    wid = cid * NS + sid

    pltpu.sync_copy(bsrc_hbm, bsrc_v)
    pltpu.sync_copy(bdst_hbm, bdst_v)

    def zero_rows(r0, sz):
      pltpu.sync_copy(zeros_hbm.at[pl.ds(0, sz)], acc_sh.at[pl.ds(r0, sz)])

    _per_tile_rows(sid, rpt, last, zero_rows)
    plsc.subcore_barrier()

    nch = base + jnp.where(wid < extra, 1, 0)

    def body(i, _):
      e0 = (wid + i * NW) * CH
      pltpu.sync_copy(src_hbm.at[pl.ds(e0, CH)], isrc_v)
      pltpu.sync_copy(bsrc_v, acc_sh.at[isrc_v], add=True)
      pltpu.sync_copy(dst_hbm.at[pl.ds(e0, CH)], idst_v)
      pltpu.sync_copy(bdst_v, acc_sh.at[idst_v], add=True)
      return 0

    lax.fori_loop(0, nch, body, 0)
    plsc.subcore_barrier()

    def writeback(r0, sz):
      nfull, rem = sz // CH, sz % CH
      for k in range(nfull):
        pltpu.sync_copy(acc_sh.at[pl.ds(r0 + k * CH, CH)], bsrc_v)
        pltpu.sync_copy(bsrc_v, out_hbm.at[cid].at[pl.ds(r0 + k * CH, CH)])
      if rem:
        pltpu.sync_copy(acc_sh.at[pl.ds(r0 + nfull * CH, rem)],
                        bsrc_v.at[pl.ds(0, rem)])
        pltpu.sync_copy(bsrc_v.at[pl.ds(0, rem)],
                        out_hbm.at[cid].at[pl.ds(r0 + nfull * CH, rem)])

    _per_tile_rows(sid, rpt, last, writeback)

  return deg_kernel


def _make_scatter_kernel(N, E, D):
  n_chunks = E // CH
  base = n_chunks // NW
  extra = n_chunks % NW
  rpt, last = _row_partition(N)

  @functools.partial(
      pl.kernel,
      out_type=jax.ShapeDtypeStruct((NC, N, D), _f32),
      mesh=_sc_mesh(),
      scratch_types=[
          pltpu.VMEM((CH,), jnp.int32),
          pltpu.VMEM((CH,), jnp.int32),
          pltpu.VMEM((CH, D), _f32),
          pltpu.VMEM_SHARED((N, D), _f32),
          pltpu.SemaphoreType.DMA,
      ],
  )
  def scatter_kernel(y_hbm, src_hbm, dst_hbm, zeros_hbm, out_hbm,
                     isrc_v, idst_v, rows_v, acc_sh, sem):
    cid = lax.axis_index("c")
    sid = lax.axis_index("s")
    wid = cid * NS + sid

    def zero_rows(r0, sz):
      pltpu.sync_copy(zeros_hbm.at[pl.ds(0, sz)], acc_sh.at[pl.ds(r0, sz)])

    _per_tile_rows(sid, rpt, last, zero_rows)
    plsc.subcore_barrier()

    nch = base + jnp.where(wid < extra, 1, 0)

    def body(i, _):
      e0 = (wid + i * NW) * CH
      pltpu.sync_copy(src_hbm.at[pl.ds(e0, CH)], isrc_v)
      pltpu.async_copy(y_hbm.at[isrc_v], rows_v, sem).wait()
      pltpu.sync_copy(dst_hbm.at[pl.ds(e0, CH)], idst_v)
      pltpu.sync_copy(rows_v, acc_sh.at[idst_v], add=True)
      return 0

    lax.fori_loop(0, nch, body, 0)
    plsc.subcore_barrier()

    def writeback(r0, sz):
      nfull, rem = sz // CH, sz % CH
      for k in range(nfull):
        pltpu.sync_copy(acc_sh.at[pl.ds(r0 + k * CH, CH)], rows_v)
        pltpu.sync_copy(rows_v, out_hbm.at[cid].at[pl.ds(r0 + k * CH, CH)])
      if rem:
        pltpu.sync_copy(acc_sh.at[pl.ds(r0 + nfull * CH, rem)],
                        rows_v.at[pl.ds(0, rem)])
        pltpu.sync_copy(rows_v.at[pl.ds(0, rem)],
                        out_hbm.at[cid].at[pl.ds(r0 + nfull * CH, rem)])

    _per_tile_rows(sid, rpt, last, writeback)

  return scatter_kernel


def _elu(z):
  return jnp.where(z > 0.0, z, jnp.exp(jnp.minimum(z, 0.0)) - 1.0)


def _dscale(dref, col):
  d = dref[0, :, col:col + 1] + dref[1, :, col:col + 1]
  return lax.rsqrt(jnp.maximum(d, 1.0))


def _tc_first_body(x_ref, w_ref, deg_ref, o_ref):
  h = jnp.dot(x_ref[...], w_ref[...], preferred_element_type=_f32)
  o_ref[...] = h * _dscale(deg_ref, 0)


def _tc_mid_body(p_ref, deg_ref, b_ref, w_ref, o_ref, *, act, dcol):
  z = (p_ref[0] + p_ref[1]) * _dscale(deg_ref, dcol) + b_ref[...]
  h = _elu(z) if act == "elu" else jnp.maximum(z, 0.0)
  y = jnp.dot(h, w_ref[...], preferred_element_type=_f32)
  o_ref[...] = y * _dscale(deg_ref, 0)


def _tc_last_body(p_ref, deg_ref, b_ref, o_ref, *, dcol):
  z = (p_ref[0] + p_ref[1]) * _dscale(deg_ref, dcol) + b_ref[...]
  o_ref[...] = _elu(z)


def kernel(X, edge_index, W_in, b_in, W_hid, b_hid, W_out, b_out):
  N, D = X.shape
  E = edge_index.shape[1]
  assert E % CH == 0 and N % 16 == 0

  src = edge_index[0]
  dst = edge_index[1]
  rpt, _ = _row_partition(N)
  zeros_blk = jnp.zeros((rpt, D), _f32)
  dcol = D // 2
  col = jnp.arange(D)
  bsrc_blk = jnp.broadcast_to((col < dcol).astype(_f32), (CH, D))
  bdst_blk = jnp.broadcast_to((col >= dcol).astype(_f32), (CH, D))

  deg_call = _make_deg_kernel(N, E, D)
  degp = deg_call(src, dst, bsrc_blk, bdst_blk, zeros_blk)

  scatter_call = _make_scatter_kernel(N, E, D)

  RB = 1000
  grid = (N // RB,)
  row_spec = pl.BlockSpec((RB, D), lambda i: (i, 0))
  part_spec = pl.BlockSpec((NC, RB, D), lambda i: (0, i, 0))
  deg_spec = part_spec
  w_spec = pl.BlockSpec((D, D), lambda i: (0, 0))
  b_spec = pl.BlockSpec((1, D), lambda i: (0, 0))
  osh = jax.ShapeDtypeStruct((N, D), _f32)

  tc_first = pl.pallas_call(
      _tc_first_body, grid=grid,
      in_specs=[row_spec, w_spec, deg_spec], out_specs=row_spec,
      out_shape=osh)
  tc_mid_elu = pl.pallas_call(
      functools.partial(_tc_mid_body, act="elu", dcol=dcol), grid=grid,
      in_specs=[part_spec, deg_spec, b_spec, w_spec],
      out_specs=row_spec, out_shape=osh)
  tc_mid_relu = pl.pallas_call(
      functools.partial(_tc_mid_body, act="relu", dcol=dcol), grid=grid,
      in_specs=[part_spec, deg_spec, b_spec, w_spec],
      out_specs=row_spec, out_shape=osh)
  tc_last = pl.pallas_call(
      functools.partial(_tc_last_body, dcol=dcol), grid=grid,
      in_specs=[part_spec, deg_spec, b_spec], out_specs=row_spec,
      out_shape=osh)

  b_in2 = b_in.reshape(1, D)
  b_hid2 = b_hid.reshape(1, D)
  b_out2 = b_out.reshape(1, D)

  y0 = tc_first(X, W_in, degp)
  p1 = scatter_call(y0, src, dst, zeros_blk)
  y1 = tc_mid_elu(p1, degp, b_in2, W_hid)
  p2 = scatter_call(y1, src, dst, zeros_blk)
  y2 = tc_mid_relu(p2, degp, b_hid2, W_out)
  p3 = scatter_call(y2, src, dst, zeros_blk)
  return tc_last(p3, degp, b_out2)


# speedup vs baseline: 13.7282x; 1.3969x over previous
import functools

import jax
import jax.numpy as jnp
from jax import lax
from jax.experimental import pallas as pl
from jax.experimental.pallas import tpu as pltpu
from jax.experimental.pallas import tpu_sc as plsc

NC = 2
NS = 16
NW = NC * NS
CH = 128

_f32 = jnp.float32


def _sc_mesh():
  return plsc.VectorSubcoreMesh(core_axis_name="c", subcore_axis_name="s")


def _row_partition(N):
  rpt = ((N // NS) + 7) // 8 * 8
  last = N - rpt * (NS - 1)
  assert 0 < last <= rpt and last % 8 == 0
  return rpt, last


def _per_tile_rows(sid, rpt, last, fn):
  @pl.when(sid < NS - 1)
  def _():
    fn(pl.multiple_of(sid * rpt, 8), rpt)

  @pl.when(sid == NS - 1)
  def _():
    fn((NS - 1) * rpt, last)


def _make_deg_kernel(N, E, D):
  n_chunks = E // CH
  base = n_chunks // NW
  extra = n_chunks % NW
  rpt, last = _row_partition(N)

  @functools.partial(
      pl.kernel,
      out_type=jax.ShapeDtypeStruct((NC, N, D), _f32),
      mesh=_sc_mesh(),
      scratch_types=[
          pltpu.VMEM((CH,), jnp.int32),
          pltpu.VMEM((CH,), jnp.int32),
          pltpu.VMEM((CH, D), _f32),
          pltpu.VMEM((CH, D), _f32),
          pltpu.VMEM_SHARED((N, D), _f32),
      ],
  )
  def deg_kernel(src_hbm, dst_hbm, bsrc_hbm, bdst_hbm, zeros_hbm, out_hbm,
                 isrc_v, idst_v, bsrc_v, bdst_v, acc_sh):
    cid = lax.axis_index("c")
    sid = lax.axis_index("s")
    wid = cid * NS + sid

    pltpu.sync_copy(bsrc_hbm, bsrc_v)
    pltpu.sync_copy(bdst_hbm, bdst_v)

    def zero_rows(r0, sz):
      pltpu.sync_copy(zeros_hbm.at[pl.ds(0, sz)], acc_sh.at[pl.ds(r0, sz)])

    _per_tile_rows(sid, rpt, last, zero_rows)
    plsc.subcore_barrier()

    nch = base + jnp.where(wid < extra, 1, 0)

    def body(i, _):
      e0 = (wid + i * NW) * CH
      pltpu.sync_copy(src_hbm.at[pl.ds(e0, CH)], isrc_v)
      pltpu.sync_copy(bsrc_v, acc_sh.at[isrc_v], add=True)
      pltpu.sync_copy(dst_hbm.at[pl.ds(e0, CH)], idst_v)
      pltpu.sync_copy(bdst_v, acc_sh.at[idst_v], add=True)
      return 0

    lax.fori_loop(0, nch, body, 0)
    plsc.subcore_barrier()

    def writeback(r0, sz):
      nfull, rem = sz // CH, sz % CH
      for k in range(nfull):
        pltpu.sync_copy(acc_sh.at[pl.ds(r0 + k * CH, CH)], bsrc_v)
        pltpu.sync_copy(bsrc_v, out_hbm.at[cid].at[pl.ds(r0 + k * CH, CH)])
      if rem:
        pltpu.sync_copy(acc_sh.at[pl.ds(r0 + nfull * CH, rem)],
                        bsrc_v.at[pl.ds(0, rem)])
        pltpu.sync_copy(bsrc_v.at[pl.ds(0, rem)],
                        out_hbm.at[cid].at[pl.ds(r0 + nfull * CH, rem)])

    _per_tile_rows(sid, rpt, last, writeback)

  return deg_kernel


def _make_scatter_kernel(N, E, D):
  n_chunks = E // CH
  base = n_chunks // NW
  extra = n_chunks % NW
  rpt, last = _row_partition(N)

  @functools.partial(
      pl.kernel,
      out_type=jax.ShapeDtypeStruct((NC, N, D), _f32),
      mesh=_sc_mesh(),
      scratch_types=[
          pltpu.VMEM((CH,), jnp.int32),
          pltpu.VMEM((CH,), jnp.int32),
          pltpu.VMEM((CH,), jnp.int32),
          pltpu.VMEM((CH,), jnp.int32),
          pltpu.VMEM((CH, D), _f32),
          pltpu.VMEM((CH, D), _f32),
          pltpu.VMEM_SHARED((N, D), _f32),
          pltpu.SemaphoreType.DMA,
          pltpu.SemaphoreType.DMA,
      ],
  )
  def scatter_kernel(y_hbm, src_hbm, dst_hbm, zeros_hbm, out_hbm,
                     isrc0, isrc1, idst0, idst1, rows0, rows1,
                     acc_sh, sem0, sem1):
    cid = lax.axis_index("c")
    sid = lax.axis_index("s")
    wid = cid * NS + sid
    slots = ((isrc0, idst0, rows0, sem0), (isrc1, idst1, rows1, sem1))

    def zero_rows(r0, sz):
      pltpu.sync_copy(zeros_hbm.at[pl.ds(0, sz)], acc_sh.at[pl.ds(r0, sz)])

    _per_tile_rows(sid, rpt, last, zero_rows)
    plsc.subcore_barrier()

    nch = base + jnp.where(wid < extra, 1, 0)

    def load_and_gather(i, isrc, idst, rows, sem):
      e0 = (wid + i * NW) * CH
      pltpu.sync_copy(src_hbm.at[pl.ds(e0, CH)], isrc)
      pltpu.sync_copy(dst_hbm.at[pl.ds(e0, CH)], idst)
      pltpu.async_copy(y_hbm.at[isrc], rows, sem)

    load_and_gather(0, *slots[0])

    def pair_body(g, _):
      for b in (0, 1):
        i = 2 * g + b
        isrc, idst, rows, sem = slots[b]
        oisrc, oidst, orows, osem = slots[1 - b]

        @pl.when(i < nch)
        def _():
          @pl.when(i + 1 < nch)
          def _():
            load_and_gather(i + 1, oisrc, oidst, orows, osem)

          pltpu.make_async_copy(y_hbm.at[isrc], rows, sem).wait()
          pltpu.sync_copy(rows, acc_sh.at[idst], add=True)

      return 0

    lax.fori_loop(0, (n_chunks + NW - 1) // NW // 2 + 1, pair_body, 0)
    plsc.subcore_barrier()

    def writeback(r0, sz):
      nfull, rem = sz // CH, sz % CH
      for k in range(nfull):
        pltpu.sync_copy(acc_sh.at[pl.ds(r0 + k * CH, CH)], rows0)
        pltpu.sync_copy(rows0, out_hbm.at[cid].at[pl.ds(r0 + k * CH, CH)])
      if rem:
        pltpu.sync_copy(acc_sh.at[pl.ds(r0 + nfull * CH, rem)],
                        rows0.at[pl.ds(0, rem)])
        pltpu.sync_copy(rows0.at[pl.ds(0, rem)],
                        out_hbm.at[cid].at[pl.ds(r0 + nfull * CH, rem)])

    _per_tile_rows(sid, rpt, last, writeback)

  return scatter_kernel


def _elu(z):
  return jnp.where(z > 0.0, z, jnp.exp(jnp.minimum(z, 0.0)) - 1.0)


def _dscale(dref, col):
  d = dref[0, :, col:col + 1] + dref[1, :, col:col + 1]
  return lax.rsqrt(jnp.maximum(d, 1.0))


def _tc_first_body(x_ref, w_ref, deg_ref, o_ref):
  h = jnp.dot(x_ref[...], w_ref[...], preferred_element_type=_f32)
  o_ref[...] = h * _dscale(deg_ref, 0)


def _tc_mid_body(p_ref, deg_ref, b_ref, w_ref, o_ref, *, act, dcol):
  z = (p_ref[0] + p_ref[1]) * _dscale(deg_ref, dcol) + b_ref[...]
  h = _elu(z) if act == "elu" else jnp.maximum(z, 0.0)
  y = jnp.dot(h, w_ref[...], preferred_element_type=_f32)
  o_ref[...] = y * _dscale(deg_ref, 0)


def _tc_last_body(p_ref, deg_ref, b_ref, o_ref, *, dcol):
  z = (p_ref[0] + p_ref[1]) * _dscale(deg_ref, dcol) + b_ref[...]
  o_ref[...] = _elu(z)


def kernel(X, edge_index, W_in, b_in, W_hid, b_hid, W_out, b_out):
  N, D = X.shape
  E = edge_index.shape[1]
  assert E % CH == 0 and N % 16 == 0

  src = edge_index[0]
  dst = edge_index[1]
  rpt, _ = _row_partition(N)
  zeros_blk = jnp.zeros((rpt, D), _f32)
  dcol = D // 2
  col = jnp.arange(D)
  bsrc_blk = jnp.broadcast_to((col < dcol).astype(_f32), (CH, D))
  bdst_blk = jnp.broadcast_to((col >= dcol).astype(_f32), (CH, D))

  deg_call = _make_deg_kernel(N, E, D)
  degp = deg_call(src, dst, bsrc_blk, bdst_blk, zeros_blk)

  scatter_call = _make_scatter_kernel(N, E, D)

  RB = 1000
  grid = (N // RB,)
  row_spec = pl.BlockSpec((RB, D), lambda i: (i, 0))
  part_spec = pl.BlockSpec((NC, RB, D), lambda i: (0, i, 0))
  deg_spec = part_spec
  w_spec = pl.BlockSpec((D, D), lambda i: (0, 0))
  b_spec = pl.BlockSpec((1, D), lambda i: (0, 0))
  osh = jax.ShapeDtypeStruct((N, D), _f32)

  tc_first = pl.pallas_call(
      _tc_first_body, grid=grid,
      in_specs=[row_spec, w_spec, deg_spec], out_specs=row_spec,
      out_shape=osh)
  tc_mid_elu = pl.pallas_call(
      functools.partial(_tc_mid_body, act="elu", dcol=dcol), grid=grid,
      in_specs=[part_spec, deg_spec, b_spec, w_spec],
      out_specs=row_spec, out_shape=osh)
  tc_mid_relu = pl.pallas_call(
      functools.partial(_tc_mid_body, act="relu", dcol=dcol), grid=grid,
      in_specs=[part_spec, deg_spec, b_spec, w_spec],
      out_specs=row_spec, out_shape=osh)
  tc_last = pl.pallas_call(
      functools.partial(_tc_last_body, dcol=dcol), grid=grid,
      in_specs=[part_spec, deg_spec, b_spec], out_specs=row_spec,
      out_shape=osh)

  b_in2 = b_in.reshape(1, D)
  b_hid2 = b_hid.reshape(1, D)
  b_out2 = b_out.reshape(1, D)

  y0 = tc_first(X, W_in, degp)
  p1 = scatter_call(y0, src, dst, zeros_blk)
  y1 = tc_mid_elu(p1, degp, b_in2, W_hid)
  p2 = scatter_call(y1, src, dst, zeros_blk)
  y2 = tc_mid_relu(p2, degp, b_hid2, W_out)
  p3 = scatter_call(y2, src, dst, zeros_blk)
  return tc_last(p3, degp, b_out2)


# speedup vs baseline: 15.1487x; 1.1035x over previous
import functools

import jax
import jax.numpy as jnp
from jax import lax
from jax.experimental import pallas as pl
from jax.experimental.pallas import tpu as pltpu
from jax.experimental.pallas import tpu_sc as plsc

NC = 2
NS = 16
NW = NC * NS
CH = 128

_f32 = jnp.float32


def _sc_mesh():
  return plsc.VectorSubcoreMesh(core_axis_name="c", subcore_axis_name="s")


def _row_partition(N):
  rpt = ((N // NS) + 7) // 8 * 8
  last = N - rpt * (NS - 1)
  assert 0 < last <= rpt and last % 8 == 0
  return rpt, last


def _per_tile_rows(sid, rpt, last, fn):
  @pl.when(sid < NS - 1)
  def _():
    fn(pl.multiple_of(sid * rpt, 8), rpt)

  @pl.when(sid == NS - 1)
  def _():
    fn((NS - 1) * rpt, last)


def _make_deg_kernel(N, E, D):
  n_chunks = E // CH
  base = n_chunks // NW
  extra = n_chunks % NW
  rpt, last = _row_partition(N)

  @functools.partial(
      pl.kernel,
      out_type=jax.ShapeDtypeStruct((NC, N, D), _f32),
      mesh=_sc_mesh(),
      scratch_types=[
          pltpu.VMEM((CH,), jnp.int32),
          pltpu.VMEM((CH,), jnp.int32),
          pltpu.VMEM((CH,), jnp.int32),
          pltpu.VMEM((CH,), jnp.int32),
          pltpu.VMEM((CH, D), _f32),
          pltpu.VMEM((CH, D), _f32),
          pltpu.VMEM_SHARED((N, D), _f32),
          pltpu.SemaphoreType.DMA,
          pltpu.SemaphoreType.DMA,
      ],
  )
  def deg_kernel(src_hbm, dst_hbm, bsrc_hbm, bdst_hbm, zeros_hbm, out_hbm,
                 isrc0, isrc1, idst0, idst1, bsrc_v, bdst_v, acc_sh,
                 sem0, sem1):
    cid = lax.axis_index("c")
    sid = lax.axis_index("s")
    wid = cid * NS + sid
    slots = ((isrc0, idst0), (isrc1, idst1))

    pltpu.sync_copy(bsrc_hbm, bsrc_v)
    pltpu.sync_copy(bdst_hbm, bdst_v)

    def zero_rows(r0, sz):
      pltpu.sync_copy(zeros_hbm.at[pl.ds(0, sz)], acc_sh.at[pl.ds(r0, sz)])

    _per_tile_rows(sid, rpt, last, zero_rows)
    plsc.subcore_barrier()

    nch = base + jnp.where(wid < extra, 1, 0)

    def load_idx(i, isrc, idst):
      e0 = (wid + i * NW) * CH
      pltpu.sync_copy(src_hbm.at[pl.ds(e0, CH)], isrc)
      pltpu.sync_copy(dst_hbm.at[pl.ds(e0, CH)], idst)

    load_idx(0, *slots[0])

    def pair_body(g, _):
      for b in (0, 1):
        i = 2 * g + b
        isrc, idst = slots[b]
        oisrc, oidst = slots[1 - b]

        @pl.when(i < nch)
        def _():
          d1 = pltpu.async_copy(bsrc_v, acc_sh.at[isrc], sem0, add=True)
          d2 = pltpu.async_copy(bdst_v, acc_sh.at[idst], sem1, add=True)

          @pl.when(i + 1 < nch)
          def _():
            load_idx(i + 1, oisrc, oidst)

          d1.wait()
          d2.wait()

      return 0

    lax.fori_loop(0, (n_chunks + NW - 1) // NW // 2 + 1, pair_body, 0)
    plsc.subcore_barrier()

    def writeback(r0, sz):
      nfull, rem = sz // CH, sz % CH
      for k in range(nfull):
        pltpu.sync_copy(acc_sh.at[pl.ds(r0 + k * CH, CH)], bsrc_v)
        pltpu.sync_copy(bsrc_v, out_hbm.at[cid].at[pl.ds(r0 + k * CH, CH)])
      if rem:
        pltpu.sync_copy(acc_sh.at[pl.ds(r0 + nfull * CH, rem)],
                        bsrc_v.at[pl.ds(0, rem)])
        pltpu.sync_copy(bsrc_v.at[pl.ds(0, rem)],
                        out_hbm.at[cid].at[pl.ds(r0 + nfull * CH, rem)])

    _per_tile_rows(sid, rpt, last, writeback)

  return deg_kernel


def _make_scatter_kernel(N, E, D):
  n_chunks = E // CH
  base = n_chunks // NW
  extra = n_chunks % NW
  rpt, last = _row_partition(N)

  @functools.partial(
      pl.kernel,
      out_type=jax.ShapeDtypeStruct((NC, N, D), _f32),
      mesh=_sc_mesh(),
      scratch_types=[
          pltpu.VMEM((CH,), jnp.int32),
          pltpu.VMEM((CH,), jnp.int32),
          pltpu.VMEM((CH,), jnp.int32),
          pltpu.VMEM((CH,), jnp.int32),
          pltpu.VMEM((CH, D), _f32),
          pltpu.VMEM((CH, D), _f32),
          pltpu.VMEM_SHARED((N, D), _f32),
          pltpu.SemaphoreType.DMA,
          pltpu.SemaphoreType.DMA,
      ],
  )
  def scatter_kernel(y_hbm, src_hbm, dst_hbm, zeros_hbm, out_hbm,
                     isrc0, isrc1, idst0, idst1, rows0, rows1,
                     acc_sh, sem0, sem1):
    cid = lax.axis_index("c")
    sid = lax.axis_index("s")
    wid = cid * NS + sid
    slots = ((isrc0, idst0, rows0, sem0), (isrc1, idst1, rows1, sem1))

    def zero_rows(r0, sz):
      pltpu.sync_copy(zeros_hbm.at[pl.ds(0, sz)], acc_sh.at[pl.ds(r0, sz)])

    _per_tile_rows(sid, rpt, last, zero_rows)
    plsc.subcore_barrier()

    nch = base + jnp.where(wid < extra, 1, 0)

    def load_and_gather(i, isrc, idst, rows, sem):
      e0 = (wid + i * NW) * CH
      pltpu.sync_copy(src_hbm.at[pl.ds(e0, CH)], isrc)
      pltpu.sync_copy(dst_hbm.at[pl.ds(e0, CH)], idst)
      pltpu.async_copy(y_hbm.at[isrc], rows, sem)

    load_and_gather(0, *slots[0])

    def pair_body(g, _):
      for b in (0, 1):
        i = 2 * g + b
        isrc, idst, rows, sem = slots[b]
        oisrc, oidst, orows, osem = slots[1 - b]

        @pl.when(i < nch)
        def _():
          @pl.when(i + 1 < nch)
          def _():
            load_and_gather(i + 1, oisrc, oidst, orows, osem)

          pltpu.make_async_copy(y_hbm.at[isrc], rows, sem).wait()
          pltpu.sync_copy(rows, acc_sh.at[idst], add=True)

      return 0

    lax.fori_loop(0, (n_chunks + NW - 1) // NW // 2 + 1, pair_body, 0)
    plsc.subcore_barrier()

    def writeback(r0, sz):
      nfull, rem = sz // CH, sz % CH
      for k in range(nfull):
        pltpu.sync_copy(acc_sh.at[pl.ds(r0 + k * CH, CH)], rows0)
        pltpu.sync_copy(rows0, out_hbm.at[cid].at[pl.ds(r0 + k * CH, CH)])
      if rem:
        pltpu.sync_copy(acc_sh.at[pl.ds(r0 + nfull * CH, rem)],
                        rows0.at[pl.ds(0, rem)])
        pltpu.sync_copy(rows0.at[pl.ds(0, rem)],
                        out_hbm.at[cid].at[pl.ds(r0 + nfull * CH, rem)])

    _per_tile_rows(sid, rpt, last, writeback)

  return scatter_kernel


def _elu(z):
  return jnp.where(z > 0.0, z, jnp.exp(jnp.minimum(z, 0.0)) - 1.0)


def _dscale(dref, col):
  d = dref[0, :, col:col + 1] + dref[1, :, col:col + 1]
  return lax.rsqrt(jnp.maximum(d, 1.0))


def _tc_first_body(x_ref, w_ref, deg_ref, o_ref):
  h = jnp.dot(x_ref[...], w_ref[...], preferred_element_type=_f32)
  o_ref[...] = h * _dscale(deg_ref, 0)


def _tc_mid_body(p_ref, deg_ref, b_ref, w_ref, o_ref, *, act, dcol):
  z = (p_ref[0] + p_ref[1]) * _dscale(deg_ref, dcol) + b_ref[...]
  h = _elu(z) if act == "elu" else jnp.maximum(z, 0.0)
  y = jnp.dot(h, w_ref[...], preferred_element_type=_f32)
  o_ref[...] = y * _dscale(deg_ref, 0)


def _tc_last_body(p_ref, deg_ref, b_ref, o_ref, *, dcol):
  z = (p_ref[0] + p_ref[1]) * _dscale(deg_ref, dcol) + b_ref[...]
  o_ref[...] = _elu(z)


def kernel(X, edge_index, W_in, b_in, W_hid, b_hid, W_out, b_out):
  N, D = X.shape
  E = edge_index.shape[1]
  assert E % CH == 0 and N % 16 == 0

  src = edge_index[0]
  dst = edge_index[1]
  rpt, _ = _row_partition(N)
  zeros_blk = jnp.zeros((rpt, D), _f32)
  dcol = D // 2
  col = jnp.arange(D)
  bsrc_blk = jnp.broadcast_to((col < dcol).astype(_f32), (CH, D))
  bdst_blk = jnp.broadcast_to((col >= dcol).astype(_f32), (CH, D))

  deg_call = _make_deg_kernel(N, E, D)
  degp = deg_call(src, dst, bsrc_blk, bdst_blk, zeros_blk)

  scatter_call = _make_scatter_kernel(N, E, D)

  RB = 1000
  grid = (N // RB,)
  row_spec = pl.BlockSpec((RB, D), lambda i: (i, 0))
  part_spec = pl.BlockSpec((NC, RB, D), lambda i: (0, i, 0))
  deg_spec = part_spec
  w_spec = pl.BlockSpec((D, D), lambda i: (0, 0))
  b_spec = pl.BlockSpec((1, D), lambda i: (0, 0))
  osh = jax.ShapeDtypeStruct((N, D), _f32)

  tc_first = pl.pallas_call(
      _tc_first_body, grid=grid,
      in_specs=[row_spec, w_spec, deg_spec], out_specs=row_spec,
      out_shape=osh)
  tc_mid_elu = pl.pallas_call(
      functools.partial(_tc_mid_body, act="elu", dcol=dcol), grid=grid,
      in_specs=[part_spec, deg_spec, b_spec, w_spec],
      out_specs=row_spec, out_shape=osh)
  tc_mid_relu = pl.pallas_call(
      functools.partial(_tc_mid_body, act="relu", dcol=dcol), grid=grid,
      in_specs=[part_spec, deg_spec, b_spec, w_spec],
      out_specs=row_spec, out_shape=osh)
  tc_last = pl.pallas_call(
      functools.partial(_tc_last_body, dcol=dcol), grid=grid,
      in_specs=[part_spec, deg_spec, b_spec], out_specs=row_spec,
      out_shape=osh)

  b_in2 = b_in.reshape(1, D)
  b_hid2 = b_hid.reshape(1, D)
  b_out2 = b_out.reshape(1, D)

  y0 = tc_first(X, W_in, degp)
  p1 = scatter_call(y0, src, dst, zeros_blk)
  y1 = tc_mid_elu(p1, degp, b_in2, W_hid)
  p2 = scatter_call(y1, src, dst, zeros_blk)
  y2 = tc_mid_relu(p2, degp, b_hid2, W_out)
  p3 = scatter_call(y2, src, dst, zeros_blk)
  return tc_last(p3, degp, b_out2)


# speedup vs baseline: 15.1563x; 1.0005x over previous
import functools

import jax
import jax.numpy as jnp
from jax import lax
from jax.experimental import pallas as pl
from jax.experimental.pallas import tpu as pltpu
from jax.experimental.pallas import tpu_sc as plsc

NC = 2
NS = 16
NW = NC * NS
CH = 128

_f32 = jnp.float32


def _sc_mesh():
  return plsc.VectorSubcoreMesh(core_axis_name="c", subcore_axis_name="s")


def _row_partition(N):
  rpt = ((N // NS) + 7) // 8 * 8
  last = N - rpt * (NS - 1)
  assert 0 < last <= rpt and last % 8 == 0
  return rpt, last


def _per_tile_rows(sid, rpt, last, fn):
  @pl.when(sid < NS - 1)
  def _():
    fn(pl.multiple_of(sid * rpt, 8), rpt)

  @pl.when(sid == NS - 1)
  def _():
    fn((NS - 1) * rpt, last)


def _make_deg_kernel(N, E, D):
  n_chunks = E // CH
  base = n_chunks // NW
  extra = n_chunks % NW
  rpt, last = _row_partition(N)

  @functools.partial(
      pl.kernel,
      out_type=jax.ShapeDtypeStruct((NC, N, D), _f32),
      mesh=_sc_mesh(),
      scratch_types=[
          pltpu.VMEM((CH,), jnp.int32),
          pltpu.VMEM((CH,), jnp.int32),
          pltpu.VMEM((CH,), jnp.int32),
          pltpu.VMEM((CH,), jnp.int32),
          pltpu.VMEM((CH, D), _f32),
          pltpu.VMEM((CH, D), _f32),
          pltpu.VMEM_SHARED((N, D), _f32),
          pltpu.SemaphoreType.DMA,
          pltpu.SemaphoreType.DMA,
      ],
  )
  def deg_kernel(src_hbm, dst_hbm, bsrc_hbm, bdst_hbm, zeros_hbm, out_hbm,
                 isrc0, isrc1, idst0, idst1, bsrc_v, bdst_v, acc_sh,
                 sem0, sem1):
    cid = lax.axis_index("c")
    sid = lax.axis_index("s")
    wid = cid * NS + sid
    slots = ((isrc0, idst0), (isrc1, idst1))

    pltpu.sync_copy(bsrc_hbm, bsrc_v)
    pltpu.sync_copy(bdst_hbm, bdst_v)

    def zero_rows(r0, sz):
      pltpu.sync_copy(zeros_hbm.at[pl.ds(0, sz)], acc_sh.at[pl.ds(r0, sz)])

    _per_tile_rows(sid, rpt, last, zero_rows)
    plsc.subcore_barrier()

    nch = base + jnp.where(wid < extra, 1, 0)

    def load_idx(i, isrc, idst):
      e0 = (wid + i * NW) * CH
      pltpu.sync_copy(src_hbm.at[pl.ds(e0, CH)], isrc)
      pltpu.sync_copy(dst_hbm.at[pl.ds(e0, CH)], idst)

    load_idx(0, *slots[0])

    def pair_body(g, _):
      for b in (0, 1):
        i = 2 * g + b
        isrc, idst = slots[b]
        oisrc, oidst = slots[1 - b]

        @pl.when(i < nch)
        def _():
          d1 = pltpu.async_copy(bsrc_v, acc_sh.at[isrc], sem0, add=True)
          d2 = pltpu.async_copy(bdst_v, acc_sh.at[idst], sem1, add=True)

          @pl.when(i + 1 < nch)
          def _():
            load_idx(i + 1, oisrc, oidst)

          d1.wait()
          d2.wait()

      return 0

    lax.fori_loop(0, (n_chunks + NW - 1) // NW // 2 + 1, pair_body, 0)
    plsc.subcore_barrier()

    def writeback(r0, sz):
      nfull, rem = sz // CH, sz % CH
      for k in range(nfull):
        pltpu.sync_copy(acc_sh.at[pl.ds(r0 + k * CH, CH)], bsrc_v)
        pltpu.sync_copy(bsrc_v, out_hbm.at[cid].at[pl.ds(r0 + k * CH, CH)])
      if rem:
        pltpu.sync_copy(acc_sh.at[pl.ds(r0 + nfull * CH, rem)],
                        bsrc_v.at[pl.ds(0, rem)])
        pltpu.sync_copy(bsrc_v.at[pl.ds(0, rem)],
                        out_hbm.at[cid].at[pl.ds(r0 + nfull * CH, rem)])

    _per_tile_rows(sid, rpt, last, writeback)

  return deg_kernel


def _make_scatter_kernel(N, E, D):
  n_chunks = E // CH
  base = n_chunks // NW
  extra = n_chunks % NW
  rpt, last = _row_partition(N)

  @functools.partial(
      pl.kernel,
      out_type=jax.ShapeDtypeStruct((NC, N, D), _f32),
      mesh=_sc_mesh(),
      scratch_types=[
          pltpu.VMEM((CH,), jnp.int32),
          pltpu.VMEM((CH,), jnp.int32),
          pltpu.VMEM((CH,), jnp.int32),
          pltpu.VMEM((CH,), jnp.int32),
          pltpu.VMEM((CH, D), _f32),
          pltpu.VMEM((CH, D), _f32),
          pltpu.VMEM_SHARED((N, D), _f32),
          pltpu.SemaphoreType.DMA,
          pltpu.SemaphoreType.DMA,
          pltpu.SemaphoreType.DMA,
          pltpu.SemaphoreType.DMA,
      ],
  )
  def scatter_kernel(y_hbm, src_hbm, dst_hbm, zeros_hbm, out_hbm,
                     isrc0, isrc1, idst0, idst1, rows0, rows1,
                     acc_sh, sem0, sem1, ssem0, ssem1):
    cid = lax.axis_index("c")
    sid = lax.axis_index("s")
    wid = cid * NS + sid
    slots = ((isrc0, idst0, rows0, sem0, ssem0),
             (isrc1, idst1, rows1, sem1, ssem1))

    def zero_rows(r0, sz):
      pltpu.sync_copy(zeros_hbm.at[pl.ds(0, sz)], acc_sh.at[pl.ds(r0, sz)])

    _per_tile_rows(sid, rpt, last, zero_rows)
    plsc.subcore_barrier()

    nch = base + jnp.where(wid < extra, 1, 0)

    def load_and_gather(i, isrc, idst, rows, sem, ssem):
      e0 = (wid + i * NW) * CH
      pltpu.sync_copy(src_hbm.at[pl.ds(e0, CH)], isrc)
      pltpu.sync_copy(dst_hbm.at[pl.ds(e0, CH)], idst)
      pltpu.async_copy(y_hbm.at[isrc], rows, sem)

    load_and_gather(0, *slots[0])

    def pair_body(g, _):
      for b in (0, 1):
        i = 2 * g + b
        isrc, idst, rows, sem, ssem = slots[b]
        oisrc, oidst, orows, osem, ossem = slots[1 - b]

        @pl.when(i < nch)
        def _():
          @pl.when(i + 1 < nch)
          def _():
            @pl.when(i >= 1)
            def _():
              pltpu.make_async_copy(orows, acc_sh.at[oidst], ossem).wait()

            load_and_gather(i + 1, oisrc, oidst, orows, osem, ossem)

          pltpu.make_async_copy(y_hbm.at[isrc], rows, sem).wait()
          pltpu.async_copy(rows, acc_sh.at[idst], ssem, add=True)

      return 0

    lax.fori_loop(0, (n_chunks + NW - 1) // NW // 2 + 1, pair_body, 0)
    pltpu.make_async_copy(rows0, acc_sh.at[idst0], ssem0).wait()
    pltpu.make_async_copy(rows1, acc_sh.at[idst1], ssem1).wait()
    plsc.subcore_barrier()

    def writeback(r0, sz):
      nfull, rem = sz // CH, sz % CH
      for k in range(nfull):
        pltpu.sync_copy(acc_sh.at[pl.ds(r0 + k * CH, CH)], rows0)
        pltpu.sync_copy(rows0, out_hbm.at[cid].at[pl.ds(r0 + k * CH, CH)])
      if rem:
        pltpu.sync_copy(acc_sh.at[pl.ds(r0 + nfull * CH, rem)],
                        rows0.at[pl.ds(0, rem)])
        pltpu.sync_copy(rows0.at[pl.ds(0, rem)],
                        out_hbm.at[cid].at[pl.ds(r0 + nfull * CH, rem)])

    _per_tile_rows(sid, rpt, last, writeback)

  return scatter_kernel


def _elu(z):
  return jnp.where(z > 0.0, z, jnp.exp(jnp.minimum(z, 0.0)) - 1.0)


def _dscale(dref, col):
  d = dref[0, :, col:col + 1] + dref[1, :, col:col + 1]
  return lax.rsqrt(jnp.maximum(d, 1.0))


def _tc_first_body(x_ref, w_ref, deg_ref, o_ref):
  h = jnp.dot(x_ref[...], w_ref[...], preferred_element_type=_f32)
  o_ref[...] = h * _dscale(deg_ref, 0)


def _tc_mid_body(p_ref, deg_ref, b_ref, w_ref, o_ref, *, act, dcol):
  z = (p_ref[0] + p_ref[1]) * _dscale(deg_ref, dcol) + b_ref[...]
  h = _elu(z) if act == "elu" else jnp.maximum(z, 0.0)
  y = jnp.dot(h, w_ref[...], preferred_element_type=_f32)
  o_ref[...] = y * _dscale(deg_ref, 0)


def _tc_last_body(p_ref, deg_ref, b_ref, o_ref, *, dcol):
  z = (p_ref[0] + p_ref[1]) * _dscale(deg_ref, dcol) + b_ref[...]
  o_ref[...] = _elu(z)


def kernel(X, edge_index, W_in, b_in, W_hid, b_hid, W_out, b_out):
  N, D = X.shape
  E = edge_index.shape[1]
  assert E % CH == 0 and N % 16 == 0

  src = edge_index[0]
  dst = edge_index[1]
  rpt, _ = _row_partition(N)
  zeros_blk = jnp.zeros((rpt, D), _f32)
  dcol = D // 2
  col = jnp.arange(D)
  bsrc_blk = jnp.broadcast_to((col < dcol).astype(_f32), (CH, D))
  bdst_blk = jnp.broadcast_to((col >= dcol).astype(_f32), (CH, D))

  deg_call = _make_deg_kernel(N, E, D)
  degp = deg_call(src, dst, bsrc_blk, bdst_blk, zeros_blk)

  scatter_call = _make_scatter_kernel(N, E, D)

  RB = 1000
  grid = (N // RB,)
  row_spec = pl.BlockSpec((RB, D), lambda i: (i, 0))
  part_spec = pl.BlockSpec((NC, RB, D), lambda i: (0, i, 0))
  deg_spec = part_spec
  w_spec = pl.BlockSpec((D, D), lambda i: (0, 0))
  b_spec = pl.BlockSpec((1, D), lambda i: (0, 0))
  osh = jax.ShapeDtypeStruct((N, D), _f32)

  tc_first = pl.pallas_call(
      _tc_first_body, grid=grid,
      in_specs=[row_spec, w_spec, deg_spec], out_specs=row_spec,
      out_shape=osh)
  tc_mid_elu = pl.pallas_call(
      functools.partial(_tc_mid_body, act="elu", dcol=dcol), grid=grid,
      in_specs=[part_spec, deg_spec, b_spec, w_spec],
      out_specs=row_spec, out_shape=osh)
  tc_mid_relu = pl.pallas_call(
      functools.partial(_tc_mid_body, act="relu", dcol=dcol), grid=grid,
      in_specs=[part_spec, deg_spec, b_spec, w_spec],
      out_specs=row_spec, out_shape=osh)
  tc_last = pl.pallas_call(
      functools.partial(_tc_last_body, dcol=dcol), grid=grid,
      in_specs=[part_spec, deg_spec, b_spec], out_specs=row_spec,
      out_shape=osh)

  b_in2 = b_in.reshape(1, D)
  b_hid2 = b_hid.reshape(1, D)
  b_out2 = b_out.reshape(1, D)

  y0 = tc_first(X, W_in, degp)
  p1 = scatter_call(y0, src, dst, zeros_blk)
  y1 = tc_mid_elu(p1, degp, b_in2, W_hid)
  p2 = scatter_call(y1, src, dst, zeros_blk)
  y2 = tc_mid_relu(p2, degp, b_hid2, W_out)
  p3 = scatter_call(y2, src, dst, zeros_blk)
  return tc_last(p3, degp, b_out2)


# speedup vs baseline: 18.7793x; 1.2390x over previous
import functools

import jax
import jax.numpy as jnp
from jax import lax
from jax.experimental import pallas as pl
from jax.experimental.pallas import tpu as pltpu
from jax.experimental.pallas import tpu_sc as plsc

NC = 2
NS = 16
NW = NC * NS
CH = 128

_f32 = jnp.float32


def _sc_mesh():
  return plsc.VectorSubcoreMesh(core_axis_name="c", subcore_axis_name="s")


def _row_partition(N):
  rpt = ((N // NS) + 7) // 8 * 8
  last = N - rpt * (NS - 1)
  assert 0 < last <= rpt and last % 8 == 0
  return rpt, last


def _per_tile_rows(sid, rpt, last, fn):
  @pl.when(sid < NS - 1)
  def _():
    fn(pl.multiple_of(sid * rpt, 8), rpt)

  @pl.when(sid == NS - 1)
  def _():
    fn((NS - 1) * rpt, last)


def _make_deg_kernel(N, E, D):
  n_chunks = E // CH
  base = n_chunks // NW
  extra = n_chunks % NW
  rpt, last = _row_partition(N)

  @functools.partial(
      pl.kernel,
      out_type=jax.ShapeDtypeStruct((NC, N, D), _f32),
      mesh=_sc_mesh(),
      scratch_types=[
          pltpu.VMEM((CH,), jnp.int32),
          pltpu.VMEM((CH,), jnp.int32),
          pltpu.VMEM((CH,), jnp.int32),
          pltpu.VMEM((CH,), jnp.int32),
          pltpu.VMEM((CH, D), _f32),
          pltpu.VMEM((CH, D), _f32),
          pltpu.VMEM_SHARED((N, D), _f32),
          pltpu.SemaphoreType.DMA,
          pltpu.SemaphoreType.DMA,
      ],
  )
  def deg_kernel(src_hbm, dst_hbm, bsrc_hbm, bdst_hbm, zeros_hbm, out_hbm,
                 isrc0, isrc1, idst0, idst1, bsrc_v, bdst_v, acc_sh,
                 sem0, sem1):
    cid = lax.axis_index("c")
    sid = lax.axis_index("s")
    wid = cid * NS + sid
    slots = ((isrc0, idst0), (isrc1, idst1))

    pltpu.sync_copy(bsrc_hbm, bsrc_v)
    pltpu.sync_copy(bdst_hbm, bdst_v)

    def zero_rows(r0, sz):
      pltpu.sync_copy(zeros_hbm.at[pl.ds(0, sz)], acc_sh.at[pl.ds(r0, sz)])

    _per_tile_rows(sid, rpt, last, zero_rows)
    plsc.subcore_barrier()

    nch = base + jnp.where(wid < extra, 1, 0)

    def load_idx(i, isrc, idst):
      e0 = (wid + i * NW) * CH
      pltpu.sync_copy(src_hbm.at[pl.ds(e0, CH)], isrc)
      pltpu.sync_copy(dst_hbm.at[pl.ds(e0, CH)], idst)

    load_idx(0, *slots[0])

    def pair_body(g, _):
      for b in (0, 1):
        i = 2 * g + b
        isrc, idst = slots[b]
        oisrc, oidst = slots[1 - b]

        @pl.when(i < nch)
        def _():
          d1 = pltpu.async_copy(bsrc_v, acc_sh.at[isrc], sem0, add=True)
          d2 = pltpu.async_copy(bdst_v, acc_sh.at[idst], sem1, add=True)

          @pl.when(i + 1 < nch)
          def _():
            load_idx(i + 1, oisrc, oidst)

          d1.wait()
          d2.wait()

      return 0

    lax.fori_loop(0, (n_chunks + NW - 1) // NW // 2 + 1, pair_body, 0)
    plsc.subcore_barrier()

    def writeback(r0, sz):
      nfull, rem = sz // CH, sz % CH
      for k in range(nfull):
        pltpu.sync_copy(acc_sh.at[pl.ds(r0 + k * CH, CH)], bsrc_v)
        pltpu.sync_copy(bsrc_v, out_hbm.at[cid].at[pl.ds(r0 + k * CH, CH)])
      if rem:
        pltpu.sync_copy(acc_sh.at[pl.ds(r0 + nfull * CH, rem)],
                        bsrc_v.at[pl.ds(0, rem)])
        pltpu.sync_copy(bsrc_v.at[pl.ds(0, rem)],
                        out_hbm.at[cid].at[pl.ds(r0 + nfull * CH, rem)])

    _per_tile_rows(sid, rpt, last, writeback)

  return deg_kernel


def _make_scatter_kernel(N, E, D):
  n_chunks = E // CH
  base = n_chunks // NW
  extra = n_chunks % NW
  rpt, last = _row_partition(N)

  @functools.partial(
      pl.kernel,
      out_type=jax.ShapeDtypeStruct((NC, N, D), _f32),
      mesh=_sc_mesh(),
      scratch_types=(
          [pltpu.VMEM((CH,), jnp.int32)] * 4
          + [pltpu.VMEM((CH,), jnp.int32)] * 4
          + [pltpu.VMEM((CH, D), _f32)] * 2
          + [pltpu.VMEM_SHARED((N, D), _f32)]
          + [pltpu.SemaphoreType.DMA] * 8
      ),
  )
  def scatter_kernel(y_hbm, src_hbm, dst_hbm, zeros_hbm, out_hbm,
                     isrc0, isrc1, isrc2, isrc3, idst0, idst1, idst2, idst3,
                     rows0, rows1, acc_sh,
                     is0, is1, is2, is3, gs0, gs1, ss0, ss1):
    cid = lax.axis_index("c")
    sid = lax.axis_index("s")
    wid = cid * NS + sid
    isrc = (isrc0, isrc1, isrc2, isrc3)
    idst = (idst0, idst1, idst2, idst3)
    isem = (is0, is1, is2, is3)
    rows = (rows0, rows1)
    gsem = (gs0, gs1)
    ssem = (ss0, ss1)

    def zero_rows(r0, sz):
      pltpu.sync_copy(zeros_hbm.at[pl.ds(0, sz)], acc_sh.at[pl.ds(r0, sz)])

    _per_tile_rows(sid, rpt, last, zero_rows)
    plsc.subcore_barrier()

    nch = base + jnp.where(wid < extra, 1, 0)

    def e0(i):
      return (wid + i * NW) * CH

    def issue_idx(i, j):
      pltpu.async_copy(src_hbm.at[pl.ds(e0(i), CH)], isrc[j], isem[j])
      pltpu.async_copy(dst_hbm.at[pl.ds(e0(i), CH)], idst[j], isem[j])

    def wait_idx(i, j):
      pltpu.make_async_copy(src_hbm.at[pl.ds(e0(i), CH)], isrc[j],
                            isem[j]).wait()
      pltpu.make_async_copy(dst_hbm.at[pl.ds(e0(i), CH)], idst[j],
                            isem[j]).wait()

    issue_idx(0, 0)
    issue_idx(1, 1)
    wait_idx(0, 0)
    pltpu.async_copy(y_hbm.at[isrc[0]], rows[0], gsem[0])

    def quad_body(g, _):
      for b4 in range(4):
        i = 4 * g + b4
        j, b = b4 % 4, b4 % 2
        jn, bn = (b4 + 1) % 4, (b4 + 1) % 2
        jp2 = (b4 + 2) % 4
        jp3 = (b4 + 3) % 4

        @pl.when(i < nch)
        def _():
          @pl.when(i + 2 < nch)
          def _():
            issue_idx(i + 2, jp2)

          @pl.when(i + 1 < nch)
          def _():
            @pl.when(i >= 1)
            def _():
              pltpu.make_async_copy(rows[bn], acc_sh.at[idst[jp3]],
                                    ssem[bn]).wait()

            wait_idx(i + 1, jn)
            pltpu.async_copy(y_hbm.at[isrc[jn]], rows[bn], gsem[bn])

          pltpu.make_async_copy(y_hbm.at[isrc[j]], rows[b], gsem[b]).wait()
          pltpu.async_copy(rows[b], acc_sh.at[idst[j]], ssem[b], add=True)

      return 0

    lax.fori_loop(0, (n_chunks + NW - 1) // NW // 4 + 1, quad_body, 0)
    pltpu.make_async_copy(rows0, acc_sh.at[idst0], ss0).wait()
    pltpu.make_async_copy(rows1, acc_sh.at[idst1], ss1).wait()
    plsc.subcore_barrier()

    def writeback(r0, sz):
      nfull, rem = sz // CH, sz % CH
      for k in range(nfull):
        pltpu.sync_copy(acc_sh.at[pl.ds(r0 + k * CH, CH)], rows0)
        pltpu.sync_copy(rows0, out_hbm.at[cid].at[pl.ds(r0 + k * CH, CH)])
      if rem:
        pltpu.sync_copy(acc_sh.at[pl.ds(r0 + nfull * CH, rem)],
                        rows0.at[pl.ds(0, rem)])
        pltpu.sync_copy(rows0.at[pl.ds(0, rem)],
                        out_hbm.at[cid].at[pl.ds(r0 + nfull * CH, rem)])

    _per_tile_rows(sid, rpt, last, writeback)

  return scatter_kernel


def _elu(z):
  return jnp.where(z > 0.0, z, jnp.exp(jnp.minimum(z, 0.0)) - 1.0)


def _dscale(dref, col):
  d = dref[0, :, col:col + 1] + dref[1, :, col:col + 1]
  return lax.rsqrt(jnp.maximum(d, 1.0))


def _tc_first_body(x_ref, w_ref, deg_ref, o_ref):
  h = jnp.dot(x_ref[...], w_ref[...], preferred_element_type=_f32)
  o_ref[...] = h * _dscale(deg_ref, 0)


def _tc_mid_body(p_ref, deg_ref, b_ref, w_ref, o_ref, *, act, dcol):
  z = (p_ref[0] + p_ref[1]) * _dscale(deg_ref, dcol) + b_ref[...]
  h = _elu(z) if act == "elu" else jnp.maximum(z, 0.0)
  y = jnp.dot(h, w_ref[...], preferred_element_type=_f32)
  o_ref[...] = y * _dscale(deg_ref, 0)


def _tc_last_body(p_ref, deg_ref, b_ref, o_ref, *, dcol):
  z = (p_ref[0] + p_ref[1]) * _dscale(deg_ref, dcol) + b_ref[...]
  o_ref[...] = _elu(z)


def kernel(X, edge_index, W_in, b_in, W_hid, b_hid, W_out, b_out):
  N, D = X.shape
  E = edge_index.shape[1]
  assert E % CH == 0 and N % 16 == 0

  src = edge_index[0]
  dst = edge_index[1]
  rpt, _ = _row_partition(N)
  zeros_blk = jnp.zeros((rpt, D), _f32)
  dcol = D // 2
  col = jnp.arange(D)
  bsrc_blk = jnp.broadcast_to((col < dcol).astype(_f32), (CH, D))
  bdst_blk = jnp.broadcast_to((col >= dcol).astype(_f32), (CH, D))

  deg_call = _make_deg_kernel(N, E, D)
  degp = deg_call(src, dst, bsrc_blk, bdst_blk, zeros_blk)

  scatter_call = _make_scatter_kernel(N, E, D)

  RB = 1000
  grid = (N // RB,)
  row_spec = pl.BlockSpec((RB, D), lambda i: (i, 0))
  part_spec = pl.BlockSpec((NC, RB, D), lambda i: (0, i, 0))
  deg_spec = part_spec
  w_spec = pl.BlockSpec((D, D), lambda i: (0, 0))
  b_spec = pl.BlockSpec((1, D), lambda i: (0, 0))
  osh = jax.ShapeDtypeStruct((N, D), _f32)

  tc_first = pl.pallas_call(
      _tc_first_body, grid=grid,
      in_specs=[row_spec, w_spec, deg_spec], out_specs=row_spec,
      out_shape=osh)
  tc_mid_elu = pl.pallas_call(
      functools.partial(_tc_mid_body, act="elu", dcol=dcol), grid=grid,
      in_specs=[part_spec, deg_spec, b_spec, w_spec],
      out_specs=row_spec, out_shape=osh)
  tc_mid_relu = pl.pallas_call(
      functools.partial(_tc_mid_body, act="relu", dcol=dcol), grid=grid,
      in_specs=[part_spec, deg_spec, b_spec, w_spec],
      out_specs=row_spec, out_shape=osh)
  tc_last = pl.pallas_call(
      functools.partial(_tc_last_body, dcol=dcol), grid=grid,
      in_specs=[part_spec, deg_spec, b_spec], out_specs=row_spec,
      out_shape=osh)

  b_in2 = b_in.reshape(1, D)
  b_hid2 = b_hid.reshape(1, D)
  b_out2 = b_out.reshape(1, D)

  y0 = tc_first(X, W_in, degp)
  p1 = scatter_call(y0, src, dst, zeros_blk)
  y1 = tc_mid_elu(p1, degp, b_in2, W_hid)
  p2 = scatter_call(y1, src, dst, zeros_blk)
  y2 = tc_mid_relu(p2, degp, b_hid2, W_out)
  p3 = scatter_call(y2, src, dst, zeros_blk)
  return tc_last(p3, degp, b_out2)


# speedup vs baseline: 18.8031x; 1.0013x over previous
import functools

import jax
import jax.numpy as jnp
from jax import lax
from jax.experimental import pallas as pl
from jax.experimental.pallas import tpu as pltpu
from jax.experimental.pallas import tpu_sc as plsc

NC = 2
NS = 16
NW = NC * NS
CH = 128

_f32 = jnp.float32


def _sc_mesh():
  return plsc.VectorSubcoreMesh(core_axis_name="c", subcore_axis_name="s")


def _row_partition(N):
  rpt = ((N // NS) + 7) // 8 * 8
  last = N - rpt * (NS - 1)
  assert 0 < last <= rpt and last % 8 == 0
  return rpt, last


def _per_tile_rows(sid, rpt, last, fn):
  @pl.when(sid < NS - 1)
  def _():
    fn(pl.multiple_of(sid * rpt, 8), rpt)

  @pl.when(sid == NS - 1)
  def _():
    fn((NS - 1) * rpt, last)


def _make_deg_kernel(N, E, D):
  n_chunks = E // CH
  base = n_chunks // NW
  extra = n_chunks % NW
  rpt, last = _row_partition(N)

  @functools.partial(
      pl.kernel,
      out_type=jax.ShapeDtypeStruct((NC, N, D), _f32),
      mesh=_sc_mesh(),
      scratch_types=(
          [pltpu.VMEM((CH,), jnp.int32)] * 4
          + [pltpu.VMEM((CH,), jnp.int32)] * 4
          + [pltpu.VMEM((CH, D), _f32)] * 2
          + [pltpu.VMEM_SHARED((N, D), _f32)]
          + [pltpu.SemaphoreType.DMA] * 6
      ),
  )
  def deg_kernel(src_hbm, dst_hbm, bsrc_hbm, bdst_hbm, zeros_hbm, out_hbm,
                 isrc0, isrc1, isrc2, isrc3, idst0, idst1, idst2, idst3,
                 bsrc_v, bdst_v, acc_sh, is0, is1, is2, is3, as0, as1):
    cid = lax.axis_index("c")
    sid = lax.axis_index("s")
    wid = cid * NS + sid
    isrc = (isrc0, isrc1, isrc2, isrc3)
    idst = (idst0, idst1, idst2, idst3)
    isem = (is0, is1, is2, is3)
    asem = (as0, as1)

    pltpu.sync_copy(bsrc_hbm, bsrc_v)
    pltpu.sync_copy(bdst_hbm, bdst_v)

    def zero_rows(r0, sz):
      pltpu.sync_copy(zeros_hbm.at[pl.ds(0, sz)], acc_sh.at[pl.ds(r0, sz)])

    _per_tile_rows(sid, rpt, last, zero_rows)
    plsc.subcore_barrier()

    nch = base + jnp.where(wid < extra, 1, 0)

    def e0(i):
      return (wid + i * NW) * CH

    def issue_idx(i, j):
      pltpu.async_copy(src_hbm.at[pl.ds(e0(i), CH)], isrc[j], isem[j])
      pltpu.async_copy(dst_hbm.at[pl.ds(e0(i), CH)], idst[j], isem[j])

    def wait_idx(i, j):
      pltpu.make_async_copy(src_hbm.at[pl.ds(e0(i), CH)], isrc[j],
                            isem[j]).wait()
      pltpu.make_async_copy(dst_hbm.at[pl.ds(e0(i), CH)], idst[j],
                            isem[j]).wait()

    issue_idx(0, 0)
    issue_idx(1, 1)

    def quad_body(g, _):
      for b4 in range(4):
        i = 4 * g + b4
        j, b = b4 % 4, b4 % 2
        jp2 = (b4 + 2) % 4

        @pl.when(i < nch)
        def _():
          @pl.when(i >= 2)
          def _():
            pltpu.make_async_copy(bsrc_v, acc_sh.at[isrc[jp2]],
                                  asem[b]).wait()
            pltpu.make_async_copy(bdst_v, acc_sh.at[idst[jp2]],
                                  asem[b]).wait()

          @pl.when(i + 2 < nch)
          def _():
            issue_idx(i + 2, jp2)

          wait_idx(i, j)
          pltpu.async_copy(bsrc_v, acc_sh.at[isrc[j]], asem[b], add=True)
          pltpu.async_copy(bdst_v, acc_sh.at[idst[j]], asem[b], add=True)

      return 0

    lax.fori_loop(0, (n_chunks + NW - 1) // NW // 4 + 1, quad_body, 0)
    for b in (0, 1):
      pltpu.make_async_copy(bsrc_v, acc_sh.at[isrc[b]], asem[b]).wait()
      pltpu.make_async_copy(bdst_v, acc_sh.at[idst[b]], asem[b]).wait()
    plsc.subcore_barrier()

    def writeback(r0, sz):
      nfull, rem = sz // CH, sz % CH
      for k in range(nfull):
        pltpu.sync_copy(acc_sh.at[pl.ds(r0 + k * CH, CH)], bsrc_v)
        pltpu.sync_copy(bsrc_v, out_hbm.at[cid].at[pl.ds(r0 + k * CH, CH)])
      if rem:
        pltpu.sync_copy(acc_sh.at[pl.ds(r0 + nfull * CH, rem)],
                        bsrc_v.at[pl.ds(0, rem)])
        pltpu.sync_copy(bsrc_v.at[pl.ds(0, rem)],
                        out_hbm.at[cid].at[pl.ds(r0 + nfull * CH, rem)])

    _per_tile_rows(sid, rpt, last, writeback)

  return deg_kernel


def _make_scatter_kernel(N, E, D):
  n_chunks = E // CH
  base = n_chunks // NW
  extra = n_chunks % NW
  rpt, last = _row_partition(N)

  @functools.partial(
      pl.kernel,
      out_type=jax.ShapeDtypeStruct((NC, N, D), _f32),
      mesh=_sc_mesh(),
      scratch_types=(
          [pltpu.VMEM((CH,), jnp.int32)] * 4
          + [pltpu.VMEM((CH,), jnp.int32)] * 4
          + [pltpu.VMEM((CH, D), _f32)] * 2
          + [pltpu.VMEM_SHARED((N, D), _f32)]
          + [pltpu.SemaphoreType.DMA] * 8
      ),
  )
  def scatter_kernel(y_hbm, src_hbm, dst_hbm, zeros_hbm, out_hbm,
                     isrc0, isrc1, isrc2, isrc3, idst0, idst1, idst2, idst3,
                     rows0, rows1, acc_sh,
                     is0, is1, is2, is3, gs0, gs1, ss0, ss1):
    cid = lax.axis_index("c")
    sid = lax.axis_index("s")
    wid = cid * NS + sid
    isrc = (isrc0, isrc1, isrc2, isrc3)
    idst = (idst0, idst1, idst2, idst3)
    isem = (is0, is1, is2, is3)
    rows = (rows0, rows1)
    gsem = (gs0, gs1)
    ssem = (ss0, ss1)

    def zero_rows(r0, sz):
      pltpu.sync_copy(zeros_hbm.at[pl.ds(0, sz)], acc_sh.at[pl.ds(r0, sz)])

    _per_tile_rows(sid, rpt, last, zero_rows)
    plsc.subcore_barrier()

    nch = base + jnp.where(wid < extra, 1, 0)

    def e0(i):
      return (wid + i * NW) * CH

    def issue_idx(i, j):
      pltpu.async_copy(src_hbm.at[pl.ds(e0(i), CH)], isrc[j], isem[j])
      pltpu.async_copy(dst_hbm.at[pl.ds(e0(i), CH)], idst[j], isem[j])

    def wait_idx(i, j):
      pltpu.make_async_copy(src_hbm.at[pl.ds(e0(i), CH)], isrc[j],
                            isem[j]).wait()
      pltpu.make_async_copy(dst_hbm.at[pl.ds(e0(i), CH)], idst[j],
                            isem[j]).wait()

    issue_idx(0, 0)
    issue_idx(1, 1)
    wait_idx(0, 0)
    pltpu.async_copy(y_hbm.at[isrc[0]], rows[0], gsem[0])

    def quad_body(g, _):
      for b4 in range(4):
        i = 4 * g + b4
        j, b = b4 % 4, b4 % 2
        jn, bn = (b4 + 1) % 4, (b4 + 1) % 2
        jp2 = (b4 + 2) % 4
        jp3 = (b4 + 3) % 4

        @pl.when(i < nch)
        def _():
          @pl.when(i + 2 < nch)
          def _():
            issue_idx(i + 2, jp2)

          @pl.when(i + 1 < nch)
          def _():
            @pl.when(i >= 1)
            def _():
              pltpu.make_async_copy(rows[bn], acc_sh.at[idst[jp3]],
                                    ssem[bn]).wait()

            wait_idx(i + 1, jn)
            pltpu.async_copy(y_hbm.at[isrc[jn]], rows[bn], gsem[bn])

          pltpu.make_async_copy(y_hbm.at[isrc[j]], rows[b], gsem[b]).wait()
          pltpu.async_copy(rows[b], acc_sh.at[idst[j]], ssem[b], add=True)

      return 0

    lax.fori_loop(0, (n_chunks + NW - 1) // NW // 4 + 1, quad_body, 0)
    pltpu.make_async_copy(rows0, acc_sh.at[idst0], ss0).wait()
    pltpu.make_async_copy(rows1, acc_sh.at[idst1], ss1).wait()
    plsc.subcore_barrier()

    def writeback(r0, sz):
      nfull, rem = sz // CH, sz % CH
      for k in range(nfull):
        pltpu.sync_copy(acc_sh.at[pl.ds(r0 + k * CH, CH)], rows0)
        pltpu.sync_copy(rows0, out_hbm.at[cid].at[pl.ds(r0 + k * CH, CH)])
      if rem:
        pltpu.sync_copy(acc_sh.at[pl.ds(r0 + nfull * CH, rem)],
                        rows0.at[pl.ds(0, rem)])
        pltpu.sync_copy(rows0.at[pl.ds(0, rem)],
                        out_hbm.at[cid].at[pl.ds(r0 + nfull * CH, rem)])

    _per_tile_rows(sid, rpt, last, writeback)

  return scatter_kernel


def _elu(z):
  return jnp.where(z > 0.0, z, jnp.exp(jnp.minimum(z, 0.0)) - 1.0)


def _dscale(dref, col):
  d = dref[0, :, col:col + 1] + dref[1, :, col:col + 1]
  return lax.rsqrt(jnp.maximum(d, 1.0))


def _tc_first_body(x_ref, w_ref, deg_ref, o_ref):
  h = jnp.dot(x_ref[...], w_ref[...], preferred_element_type=_f32)
  o_ref[...] = h * _dscale(deg_ref, 0)


def _tc_mid_body(p_ref, deg_ref, b_ref, w_ref, o_ref, *, act, dcol):
  z = (p_ref[0] + p_ref[1]) * _dscale(deg_ref, dcol) + b_ref[...]
  h = _elu(z) if act == "elu" else jnp.maximum(z, 0.0)
  y = jnp.dot(h, w_ref[...], preferred_element_type=_f32)
  o_ref[...] = y * _dscale(deg_ref, 0)


def _tc_last_body(p_ref, deg_ref, b_ref, o_ref, *, dcol):
  z = (p_ref[0] + p_ref[1]) * _dscale(deg_ref, dcol) + b_ref[...]
  o_ref[...] = _elu(z)


def kernel(X, edge_index, W_in, b_in, W_hid, b_hid, W_out, b_out):
  N, D = X.shape
  E = edge_index.shape[1]
  assert E % CH == 0 and N % 16 == 0

  src = edge_index[0]
  dst = edge_index[1]
  rpt, _ = _row_partition(N)
  zeros_blk = jnp.zeros((rpt, D), _f32)
  dcol = D // 2
  col = jnp.arange(D)
  bsrc_blk = jnp.broadcast_to((col < dcol).astype(_f32), (CH, D))
  bdst_blk = jnp.broadcast_to((col >= dcol).astype(_f32), (CH, D))

  deg_call = _make_deg_kernel(N, E, D)
  degp = deg_call(src, dst, bsrc_blk, bdst_blk, zeros_blk)

  scatter_call = _make_scatter_kernel(N, E, D)

  RB = 1000
  grid = (N // RB,)
  row_spec = pl.BlockSpec((RB, D), lambda i: (i, 0))
  part_spec = pl.BlockSpec((NC, RB, D), lambda i: (0, i, 0))
  deg_spec = part_spec
  w_spec = pl.BlockSpec((D, D), lambda i: (0, 0))
  b_spec = pl.BlockSpec((1, D), lambda i: (0, 0))
  osh = jax.ShapeDtypeStruct((N, D), _f32)

  tc_first = pl.pallas_call(
      _tc_first_body, grid=grid,
      in_specs=[row_spec, w_spec, deg_spec], out_specs=row_spec,
      out_shape=osh)
  tc_mid_elu = pl.pallas_call(
      functools.partial(_tc_mid_body, act="elu", dcol=dcol), grid=grid,
      in_specs=[part_spec, deg_spec, b_spec, w_spec],
      out_specs=row_spec, out_shape=osh)
  tc_mid_relu = pl.pallas_call(
      functools.partial(_tc_mid_body, act="relu", dcol=dcol), grid=grid,
      in_specs=[part_spec, deg_spec, b_spec, w_spec],
      out_specs=row_spec, out_shape=osh)
  tc_last = pl.pallas_call(
      functools.partial(_tc_last_body, dcol=dcol), grid=grid,
      in_specs=[part_spec, deg_spec, b_spec], out_specs=row_spec,
      out_shape=osh)

  b_in2 = b_in.reshape(1, D)
  b_hid2 = b_hid.reshape(1, D)
  b_out2 = b_out.reshape(1, D)

  y0 = tc_first(X, W_in, degp)
  p1 = scatter_call(y0, src, dst, zeros_blk)
  y1 = tc_mid_elu(p1, degp, b_in2, W_hid)
  p2 = scatter_call(y1, src, dst, zeros_blk)
  y2 = tc_mid_relu(p2, degp, b_hid2, W_out)
  p3 = scatter_call(y2, src, dst, zeros_blk)
  return tc_last(p3, degp, b_out2)


# speedup vs baseline: 18.8763x; 1.0039x over previous
import functools

import jax
import jax.numpy as jnp
from jax import lax
from jax.experimental import pallas as pl
from jax.experimental.pallas import tpu as pltpu
from jax.experimental.pallas import tpu_sc as plsc

NC = 2
NS = 16
NW = NC * NS
CH = 128

_f32 = jnp.float32


def _sc_mesh():
  return plsc.VectorSubcoreMesh(core_axis_name="c", subcore_axis_name="s")


def _row_partition(N):
  rpt = ((N // NS) + 7) // 8 * 8
  last = N - rpt * (NS - 1)
  assert 0 < last <= rpt and last % 8 == 0
  return rpt, last


def _per_tile_rows(sid, rpt, last, fn):
  @pl.when(sid < NS - 1)
  def _():
    fn(pl.multiple_of(sid * rpt, 8), rpt)

  @pl.when(sid == NS - 1)
  def _():
    fn((NS - 1) * rpt, last)


def _make_deg_kernel(N, E, D):
  n_chunks = E // CH
  base = n_chunks // NW
  extra = n_chunks % NW
  rpt, last = _row_partition(N)

  @functools.partial(
      pl.kernel,
      out_type=jax.ShapeDtypeStruct((NC, N, D), _f32),
      mesh=_sc_mesh(),
      scratch_types=(
          [pltpu.VMEM((CH,), jnp.int32)] * 4
          + [pltpu.VMEM((CH,), jnp.int32)] * 4
          + [pltpu.VMEM((CH, D), _f32)] * 2
          + [pltpu.VMEM_SHARED((N, D), _f32)]
          + [pltpu.SemaphoreType.DMA] * 6
      ),
  )
  def deg_kernel(src_hbm, dst_hbm, bsrc_hbm, bdst_hbm, zeros_hbm, out_hbm,
                 isrc0, isrc1, isrc2, isrc3, idst0, idst1, idst2, idst3,
                 bsrc_v, bdst_v, acc_sh, is0, is1, is2, is3, as0, as1):
    cid = lax.axis_index("c")
    sid = lax.axis_index("s")
    wid = cid * NS + sid
    isrc = (isrc0, isrc1, isrc2, isrc3)
    idst = (idst0, idst1, idst2, idst3)
    isem = (is0, is1, is2, is3)
    asem = (as0, as1)

    pltpu.sync_copy(bsrc_hbm, bsrc_v)
    pltpu.sync_copy(bdst_hbm, bdst_v)

    def zero_rows(r0, sz):
      pltpu.sync_copy(zeros_hbm.at[pl.ds(0, sz)], acc_sh.at[pl.ds(r0, sz)])

    _per_tile_rows(sid, rpt, last, zero_rows)
    plsc.subcore_barrier()

    nch = base + jnp.where(wid < extra, 1, 0)

    def e0(i):
      return (wid + i * NW) * CH

    def issue_idx(i, j):
      pltpu.async_copy(src_hbm.at[pl.ds(e0(i), CH)], isrc[j], isem[j])
      pltpu.async_copy(dst_hbm.at[pl.ds(e0(i), CH)], idst[j], isem[j])

    def wait_idx(i, j):
      pltpu.make_async_copy(src_hbm.at[pl.ds(e0(i), CH)], isrc[j],
                            isem[j]).wait()
      pltpu.make_async_copy(dst_hbm.at[pl.ds(e0(i), CH)], idst[j],
                            isem[j]).wait()

    issue_idx(0, 0)
    issue_idx(1, 1)

    def quad_body(g, _):
      for b4 in range(4):
        i = 4 * g + b4
        j, b = b4 % 4, b4 % 2
        jp2 = (b4 + 2) % 4

        @pl.when(i < nch)
        def _():
          @pl.when(i >= 2)
          def _():
            pltpu.make_async_copy(bsrc_v, acc_sh.at[isrc[jp2]],
                                  asem[b]).wait()
            pltpu.make_async_copy(bdst_v, acc_sh.at[idst[jp2]],
                                  asem[b]).wait()

          @pl.when(i + 2 < nch)
          def _():
            issue_idx(i + 2, jp2)

          wait_idx(i, j)
          pltpu.async_copy(bsrc_v, acc_sh.at[isrc[j]], asem[b], add=True)
          pltpu.async_copy(bdst_v, acc_sh.at[idst[j]], asem[b], add=True)

      return 0

    lax.fori_loop(0, (n_chunks + NW - 1) // NW // 4 + 1, quad_body, 0)
    for b in (0, 1):
      pltpu.make_async_copy(bsrc_v, acc_sh.at[isrc[b]], asem[b]).wait()
      pltpu.make_async_copy(bdst_v, acc_sh.at[idst[b]], asem[b]).wait()
    plsc.subcore_barrier()

    def writeback(r0, sz):
      pltpu.sync_copy(acc_sh.at[pl.ds(r0, sz)],
                      out_hbm.at[cid].at[pl.ds(r0, sz)])

    _per_tile_rows(sid, rpt, last, writeback)

  return deg_kernel


def _make_scatter_kernel(N, E, D):
  n_chunks = E // CH
  base = n_chunks // NW
  extra = n_chunks % NW
  rpt, last = _row_partition(N)

  @functools.partial(
      pl.kernel,
      out_type=jax.ShapeDtypeStruct((NC, N, D), _f32),
      mesh=_sc_mesh(),
      scratch_types=(
          [pltpu.VMEM((CH,), jnp.int32)] * 4
          + [pltpu.VMEM((CH,), jnp.int32)] * 4
          + [pltpu.VMEM((CH, D), _f32)] * 2
          + [pltpu.VMEM_SHARED((N, D), _f32)]
          + [pltpu.SemaphoreType.DMA] * 8
      ),
  )
  def scatter_kernel(y_hbm, src_hbm, dst_hbm, zeros_hbm, out_hbm,
                     isrc0, isrc1, isrc2, isrc3, idst0, idst1, idst2, idst3,
                     rows0, rows1, acc_sh,
                     is0, is1, is2, is3, gs0, gs1, ss0, ss1):
    cid = lax.axis_index("c")
    sid = lax.axis_index("s")
    wid = cid * NS + sid
    isrc = (isrc0, isrc1, isrc2, isrc3)
    idst = (idst0, idst1, idst2, idst3)
    isem = (is0, is1, is2, is3)
    rows = (rows0, rows1)
    gsem = (gs0, gs1)
    ssem = (ss0, ss1)

    def zero_rows(r0, sz):
      pltpu.sync_copy(zeros_hbm.at[pl.ds(0, sz)], acc_sh.at[pl.ds(r0, sz)])

    _per_tile_rows(sid, rpt, last, zero_rows)
    plsc.subcore_barrier()

    nch = base + jnp.where(wid < extra, 1, 0)

    def e0(i):
      return (wid + i * NW) * CH

    def issue_idx(i, j):
      pltpu.async_copy(src_hbm.at[pl.ds(e0(i), CH)], isrc[j], isem[j])
      pltpu.async_copy(dst_hbm.at[pl.ds(e0(i), CH)], idst[j], isem[j])

    def wait_idx(i, j):
      pltpu.make_async_copy(src_hbm.at[pl.ds(e0(i), CH)], isrc[j],
                            isem[j]).wait()
      pltpu.make_async_copy(dst_hbm.at[pl.ds(e0(i), CH)], idst[j],
                            isem[j]).wait()

    issue_idx(0, 0)
    issue_idx(1, 1)
    wait_idx(0, 0)
    pltpu.async_copy(y_hbm.at[isrc[0]], rows[0], gsem[0])

    def quad_body(g, _):
      for b4 in range(4):
        i = 4 * g + b4
        j, b = b4 % 4, b4 % 2
        jn, bn = (b4 + 1) % 4, (b4 + 1) % 2
        jp2 = (b4 + 2) % 4
        jp3 = (b4 + 3) % 4

        @pl.when(i < nch)
        def _():
          @pl.when(i + 2 < nch)
          def _():
            issue_idx(i + 2, jp2)

          @pl.when(i + 1 < nch)
          def _():
            @pl.when(i >= 1)
            def _():
              pltpu.make_async_copy(rows[bn], acc_sh.at[idst[jp3]],
                                    ssem[bn]).wait()

            wait_idx(i + 1, jn)
            pltpu.async_copy(y_hbm.at[isrc[jn]], rows[bn], gsem[bn])

          pltpu.make_async_copy(y_hbm.at[isrc[j]], rows[b], gsem[b]).wait()
          pltpu.async_copy(rows[b], acc_sh.at[idst[j]], ssem[b], add=True)

      return 0

    lax.fori_loop(0, (n_chunks + NW - 1) // NW // 4 + 1, quad_body, 0)
    pltpu.make_async_copy(rows0, acc_sh.at[idst0], ss0).wait()
    pltpu.make_async_copy(rows1, acc_sh.at[idst1], ss1).wait()
    plsc.subcore_barrier()

    def writeback(r0, sz):
      pltpu.sync_copy(acc_sh.at[pl.ds(r0, sz)],
                      out_hbm.at[cid].at[pl.ds(r0, sz)])

    _per_tile_rows(sid, rpt, last, writeback)

  return scatter_kernel


def _elu(z):
  return jnp.where(z > 0.0, z, jnp.exp(jnp.minimum(z, 0.0)) - 1.0)


def _dscale(dref, col):
  d = dref[0, :, col:col + 1] + dref[1, :, col:col + 1]
  return lax.rsqrt(jnp.maximum(d, 1.0))


def _tc_first_body(x_ref, w_ref, deg_ref, o_ref):
  h = jnp.dot(x_ref[...], w_ref[...], preferred_element_type=_f32)
  o_ref[...] = h * _dscale(deg_ref, 0)


def _tc_mid_body(p_ref, deg_ref, b_ref, w_ref, o_ref, *, act, dcol):
  z = (p_ref[0] + p_ref[1]) * _dscale(deg_ref, dcol) + b_ref[...]
  h = _elu(z) if act == "elu" else jnp.maximum(z, 0.0)
  y = jnp.dot(h, w_ref[...], preferred_element_type=_f32)
  o_ref[...] = y * _dscale(deg_ref, 0)


def _tc_last_body(p_ref, deg_ref, b_ref, o_ref, *, dcol):
  z = (p_ref[0] + p_ref[1]) * _dscale(deg_ref, dcol) + b_ref[...]
  o_ref[...] = _elu(z)


def kernel(X, edge_index, W_in, b_in, W_hid, b_hid, W_out, b_out):
  N, D = X.shape
  E = edge_index.shape[1]
  assert E % CH == 0 and N % 16 == 0

  src = edge_index[0]
  dst = edge_index[1]
  rpt, _ = _row_partition(N)
  zeros_blk = jnp.zeros((rpt, D), _f32)
  dcol = D // 2
  col = jnp.arange(D)
  bsrc_blk = jnp.broadcast_to((col < dcol).astype(_f32), (CH, D))
  bdst_blk = jnp.broadcast_to((col >= dcol).astype(_f32), (CH, D))

  deg_call = _make_deg_kernel(N, E, D)
  degp = deg_call(src, dst, bsrc_blk, bdst_blk, zeros_blk)

  scatter_call = _make_scatter_kernel(N, E, D)

  RB = 1000
  grid = (N // RB,)
  row_spec = pl.BlockSpec((RB, D), lambda i: (i, 0))
  part_spec = pl.BlockSpec((NC, RB, D), lambda i: (0, i, 0))
  deg_spec = part_spec
  w_spec = pl.BlockSpec((D, D), lambda i: (0, 0))
  b_spec = pl.BlockSpec((1, D), lambda i: (0, 0))
  osh = jax.ShapeDtypeStruct((N, D), _f32)

  tc_first = pl.pallas_call(
      _tc_first_body, grid=grid,
      in_specs=[row_spec, w_spec, deg_spec], out_specs=row_spec,
      out_shape=osh)
  tc_mid_elu = pl.pallas_call(
      functools.partial(_tc_mid_body, act="elu", dcol=dcol), grid=grid,
      in_specs=[part_spec, deg_spec, b_spec, w_spec],
      out_specs=row_spec, out_shape=osh)
  tc_mid_relu = pl.pallas_call(
      functools.partial(_tc_mid_body, act="relu", dcol=dcol), grid=grid,
      in_specs=[part_spec, deg_spec, b_spec, w_spec],
      out_specs=row_spec, out_shape=osh)
  tc_last = pl.pallas_call(
      functools.partial(_tc_last_body, dcol=dcol), grid=grid,
      in_specs=[part_spec, deg_spec, b_spec], out_specs=row_spec,
      out_shape=osh)

  b_in2 = b_in.reshape(1, D)
  b_hid2 = b_hid.reshape(1, D)
  b_out2 = b_out.reshape(1, D)

  y0 = tc_first(X, W_in, degp)
  p1 = scatter_call(y0, src, dst, zeros_blk)
  y1 = tc_mid_elu(p1, degp, b_in2, W_hid)
  p2 = scatter_call(y1, src, dst, zeros_blk)
  y2 = tc_mid_relu(p2, degp, b_hid2, W_out)
  p3 = scatter_call(y2, src, dst, zeros_blk)
  return tc_last(p3, degp, b_out2)


# speedup vs baseline: 19.1786x; 1.0160x over previous
import functools

import jax
import jax.numpy as jnp
from jax import lax
from jax.experimental import pallas as pl
from jax.experimental.pallas import tpu as pltpu
from jax.experimental.pallas import tpu_sc as plsc

NC = 2
NS = 16
NW = NC * NS
CH = 128

_f32 = jnp.float32


def _sc_mesh():
  return plsc.VectorSubcoreMesh(core_axis_name="c", subcore_axis_name="s")


def _row_partition(N):
  rpt = ((N // NS) + 7) // 8 * 8
  last = N - rpt * (NS - 1)
  assert 0 < last <= rpt and last % 8 == 0
  return rpt, last


def _per_tile_rows(sid, rpt, last, fn):
  @pl.when(sid < NS - 1)
  def _():
    fn(pl.multiple_of(sid * rpt, 8), rpt)

  @pl.when(sid == NS - 1)
  def _():
    fn((NS - 1) * rpt, last)


def _make_deg_kernel(N, E, D):
  n_chunks = E // CH
  base = n_chunks // NW
  extra = n_chunks % NW
  rpt, last = _row_partition(N)

  @functools.partial(
      pl.kernel,
      out_type=jax.ShapeDtypeStruct((NC, N, D), _f32),
      mesh=_sc_mesh(),
      scratch_types=(
          [pltpu.VMEM((CH,), jnp.int32)] * 4
          + [pltpu.VMEM((CH,), jnp.int32)] * 4
          + [pltpu.VMEM((CH, D), _f32)] * 2
          + [pltpu.VMEM_SHARED((N, D), _f32)]
          + [pltpu.SemaphoreType.DMA] * 6
      ),
  )
  def deg_kernel(src_hbm, dst_hbm, bsrc_hbm, bdst_hbm, zeros_hbm, out_hbm,
                 isrc0, isrc1, isrc2, isrc3, idst0, idst1, idst2, idst3,
                 bsrc_v, bdst_v, acc_sh, is0, is1, is2, is3, as0, as1):
    cid = lax.axis_index("c")
    sid = lax.axis_index("s")
    wid = cid * NS + sid
    isrc = (isrc0, isrc1, isrc2, isrc3)
    idst = (idst0, idst1, idst2, idst3)
    isem = (is0, is1, is2, is3)
    asem = (as0, as1)

    pltpu.sync_copy(bsrc_hbm, bsrc_v)
    pltpu.sync_copy(bdst_hbm, bdst_v)

    def zero_rows(r0, sz):
      pltpu.sync_copy(zeros_hbm.at[pl.ds(0, sz)], acc_sh.at[pl.ds(r0, sz)])

    _per_tile_rows(sid, rpt, last, zero_rows)
    plsc.subcore_barrier()

    nch = base + jnp.where(wid < extra, 1, 0)

    def e0(i):
      return (wid + i * NW) * CH

    def issue_idx(i, j):
      pltpu.async_copy(src_hbm.at[pl.ds(e0(i), CH)], isrc[j], isem[j])
      pltpu.async_copy(dst_hbm.at[pl.ds(e0(i), CH)], idst[j], isem[j])

    def wait_idx(i, j):
      pltpu.make_async_copy(src_hbm.at[pl.ds(e0(i), CH)], isrc[j],
                            isem[j]).wait()
      pltpu.make_async_copy(dst_hbm.at[pl.ds(e0(i), CH)], idst[j],
                            isem[j]).wait()

    issue_idx(0, 0)
    issue_idx(1, 1)

    def quad_body(g, _):
      for b4 in range(4):
        i = 4 * g + b4
        j, b = b4 % 4, b4 % 2
        jp2 = (b4 + 2) % 4

        @pl.when(i < nch)
        def _():
          @pl.when(i >= 2)
          def _():
            pltpu.make_async_copy(bsrc_v, acc_sh.at[isrc[jp2]],
                                  asem[b]).wait()
            pltpu.make_async_copy(bdst_v, acc_sh.at[idst[jp2]],
                                  asem[b]).wait()

          @pl.when(i + 2 < nch)
          def _():
            issue_idx(i + 2, jp2)

          wait_idx(i, j)
          pltpu.async_copy(bsrc_v, acc_sh.at[isrc[j]], asem[b], add=True)
          pltpu.async_copy(bdst_v, acc_sh.at[idst[j]], asem[b], add=True)

      return 0

    lax.fori_loop(0, (n_chunks + NW - 1) // NW // 4 + 1, quad_body, 0)
    for b in (0, 1):
      pltpu.make_async_copy(bsrc_v, acc_sh.at[isrc[b]], asem[b]).wait()
      pltpu.make_async_copy(bdst_v, acc_sh.at[idst[b]], asem[b]).wait()
    plsc.subcore_barrier()

    def writeback(r0, sz):
      pltpu.sync_copy(acc_sh.at[pl.ds(r0, sz)],
                      out_hbm.at[cid].at[pl.ds(r0, sz)])

    _per_tile_rows(sid, rpt, last, writeback)

  return deg_kernel


def _make_scatter_kernel(N, E, D):
  n_chunks = E // CH
  base = n_chunks // NW
  extra = n_chunks % NW
  rpt, last = _row_partition(N)

  @functools.partial(
      pl.kernel,
      out_type=jax.ShapeDtypeStruct((NC, N, D), _f32),
      mesh=_sc_mesh(),
      scratch_types=(
          [pltpu.VMEM((CH,), jnp.int32)] * 4
          + [pltpu.VMEM((CH,), jnp.int32)] * 4
          + [pltpu.VMEM((CH, D), _f32)] * 2
          + [pltpu.VMEM_SHARED((N, D), _f32)]
          + [pltpu.SemaphoreType.DMA] * 8
      ),
  )
  def scatter_kernel(y_hbm, src_hbm, dst_hbm, zeros_hbm, out_hbm,
                     isrc0, isrc1, isrc2, isrc3, idst0, idst1, idst2, idst3,
                     rows0, rows1, acc_sh,
                     is0, is1, is2, is3, gs0, gs1, ss0, ss1):
    cid = lax.axis_index("c")
    sid = lax.axis_index("s")
    wid = cid * NS + sid
    isrc = (isrc0, isrc1, isrc2, isrc3)
    idst = (idst0, idst1, idst2, idst3)
    isem = (is0, is1, is2, is3)
    rows = (rows0, rows1)
    gsem = (gs0, gs1)
    ssem = (ss0, ss1)

    def zero_rows(r0, sz):
      pltpu.sync_copy(zeros_hbm.at[pl.ds(0, sz)], acc_sh.at[pl.ds(r0, sz)])

    _per_tile_rows(sid, rpt, last, zero_rows)
    plsc.subcore_barrier()

    nch = base + jnp.where(wid < extra, 1, 0)

    def e0(i):
      return (wid + i * NW) * CH

    def issue_idx(i, j):
      pltpu.async_copy(src_hbm.at[pl.ds(e0(i), CH)], isrc[j], isem[j])
      pltpu.async_copy(dst_hbm.at[pl.ds(e0(i), CH)], idst[j], isem[j])

    def wait_idx(i, j):
      pltpu.make_async_copy(src_hbm.at[pl.ds(e0(i), CH)], isrc[j],
                            isem[j]).wait()
      pltpu.make_async_copy(dst_hbm.at[pl.ds(e0(i), CH)], idst[j],
                            isem[j]).wait()

    issue_idx(0, 0)
    issue_idx(1, 1)
    wait_idx(0, 0)
    pltpu.async_copy(y_hbm.at[isrc[0]], rows[0], gsem[0])

    def quad_body(g, _):
      for b4 in range(4):
        i = 4 * g + b4
        j, b = b4 % 4, b4 % 2
        jn, bn = (b4 + 1) % 4, (b4 + 1) % 2
        jp2 = (b4 + 2) % 4
        jp3 = (b4 + 3) % 4

        @pl.when(i < nch)
        def _():
          @pl.when(i + 2 < nch)
          def _():
            issue_idx(i + 2, jp2)

          @pl.when(i + 1 < nch)
          def _():
            @pl.when(i >= 1)
            def _():
              pltpu.make_async_copy(rows[bn], acc_sh.at[idst[jp3]],
                                    ssem[bn]).wait()

            wait_idx(i + 1, jn)
            pltpu.async_copy(y_hbm.at[isrc[jn]], rows[bn], gsem[bn])

          pltpu.make_async_copy(y_hbm.at[isrc[j]], rows[b], gsem[b]).wait()
          pltpu.async_copy(rows[b], acc_sh.at[idst[j]], ssem[b], add=True)

      return 0

    lax.fori_loop(0, (n_chunks + NW - 1) // NW // 4 + 1, quad_body, 0)
    pltpu.make_async_copy(rows0, acc_sh.at[idst0], ss0).wait()
    pltpu.make_async_copy(rows1, acc_sh.at[idst1], ss1).wait()
    plsc.subcore_barrier()

    def writeback(r0, sz):
      pltpu.sync_copy(acc_sh.at[pl.ds(r0, sz)],
                      out_hbm.at[cid].at[pl.ds(r0, sz)])

    _per_tile_rows(sid, rpt, last, writeback)

  return scatter_kernel


def _elu(z):
  return jnp.where(z > 0.0, z, jnp.exp(jnp.minimum(z, 0.0)) - 1.0)


def _dscale(dref, col):
  d = dref[0, :, col:col + 1] + dref[1, :, col:col + 1]
  return lax.rsqrt(jnp.maximum(d, 1.0))


def _tc_first_body(x_ref, w_ref, deg_ref, o_ref):
  h = jnp.dot(x_ref[...], w_ref[...], preferred_element_type=_f32)
  o_ref[...] = h * _dscale(deg_ref, 0)


def _tc_mid_body(p_ref, deg_ref, b_ref, w_ref, o_ref, *, act, dcol):
  z = (p_ref[0] + p_ref[1]) * _dscale(deg_ref, dcol) + b_ref[...]
  h = _elu(z) if act == "elu" else jnp.maximum(z, 0.0)
  y = jnp.dot(h, w_ref[...], preferred_element_type=_f32)
  o_ref[...] = y * _dscale(deg_ref, 0)


def _tc_last_body(p_ref, deg_ref, b_ref, o_ref, *, dcol):
  z = (p_ref[0] + p_ref[1]) * _dscale(deg_ref, dcol) + b_ref[...]
  o_ref[...] = _elu(z)


def kernel(X, edge_index, W_in, b_in, W_hid, b_hid, W_out, b_out):
  N, D = X.shape
  E = edge_index.shape[1]
  assert E % CH == 0 and N % 16 == 0

  src = edge_index[0]
  dst = edge_index[1]
  rpt, _ = _row_partition(N)
  zeros_blk = jnp.zeros((rpt, D), _f32)
  dcol = D // 2
  col = jnp.arange(D)
  bsrc_blk = jnp.broadcast_to((col < dcol).astype(_f32), (CH, D))
  bdst_blk = jnp.broadcast_to((col >= dcol).astype(_f32), (CH, D))

  deg_call = _make_deg_kernel(N, E, D)
  degp = deg_call(src, dst, bsrc_blk, bdst_blk, zeros_blk)

  scatter_call = _make_scatter_kernel(N, E, D)

  RB = 2000
  grid = (N // RB,)
  row_spec = pl.BlockSpec((RB, D), lambda i: (i, 0))
  part_spec = pl.BlockSpec((NC, RB, D), lambda i: (0, i, 0))
  deg_spec = part_spec
  w_spec = pl.BlockSpec((D, D), lambda i: (0, 0))
  b_spec = pl.BlockSpec((1, D), lambda i: (0, 0))
  osh = jax.ShapeDtypeStruct((N, D), _f32)

  tc_first = pl.pallas_call(
      _tc_first_body, grid=grid,
      in_specs=[row_spec, w_spec, deg_spec], out_specs=row_spec,
      out_shape=osh)
  tc_mid_elu = pl.pallas_call(
      functools.partial(_tc_mid_body, act="elu", dcol=dcol), grid=grid,
      in_specs=[part_spec, deg_spec, b_spec, w_spec],
      out_specs=row_spec, out_shape=osh)
  tc_mid_relu = pl.pallas_call(
      functools.partial(_tc_mid_body, act="relu", dcol=dcol), grid=grid,
      in_specs=[part_spec, deg_spec, b_spec, w_spec],
      out_specs=row_spec, out_shape=osh)
  tc_last = pl.pallas_call(
      functools.partial(_tc_last_body, dcol=dcol), grid=grid,
      in_specs=[part_spec, deg_spec, b_spec], out_specs=row_spec,
      out_shape=osh)

  b_in2 = b_in.reshape(1, D)
  b_hid2 = b_hid.reshape(1, D)
  b_out2 = b_out.reshape(1, D)

  y0 = tc_first(X, W_in, degp)
  p1 = scatter_call(y0, src, dst, zeros_blk)
  y1 = tc_mid_elu(p1, degp, b_in2, W_hid)
  p2 = scatter_call(y1, src, dst, zeros_blk)
  y2 = tc_mid_relu(p2, degp, b_hid2, W_out)
  p3 = scatter_call(y2, src, dst, zeros_blk)
  return tc_last(p3, degp, b_out2)
